# baseline (speedup 1.0000x reference)
import functools

import jax
import jax.numpy as jnp
from jax import lax
from jax.experimental import pallas as pl
from jax.experimental.pallas import tpu as pltpu
from jax.experimental.pallas import tpu_sc as plsc

_CHUNK_ROWS = 32
_NBUF = 4


def kernel(seq_len, table):
    del seq_len
    rows, dim = table.shape
    info = plsc.get_sparse_core_info()
    num_workers = info.num_cores * info.num_subcores
    rows_per_worker = rows // num_workers
    nchunk = rows_per_worker // _CHUNK_ROWS

    mesh = plsc.VectorSubcoreMesh(core_axis_name="c", subcore_axis_name="s")

    nbuf = _NBUF

    @functools.partial(
        pl.kernel,
        mesh=mesh,
        out_type=jax.ShapeDtypeStruct((rows, dim), table.dtype),
        scratch_types=(
            [pltpu.VMEM((_CHUNK_ROWS, dim), jnp.float32) for _ in range(nbuf)]
            + [pltpu.SemaphoreType.DMA for _ in range(2 * nbuf)]
        ),
    )
    def sc_copy(table_hbm, out_hbm, *scratch):
        bufs = scratch[:nbuf]
        load_sems = scratch[nbuf : 2 * nbuf]
        store_sems = scratch[2 * nbuf :]
        wid = lax.axis_index("s") * info.num_cores + lax.axis_index("c")
        base = wid * rows_per_worker

        def start_load(c, b):
            return pltpu.async_copy(
                table_hbm.at[pl.ds(base + c * _CHUNK_ROWS, _CHUNK_ROWS)],
                bufs[b],
                load_sems[b],
            )

        def start_store(c, b):
            return pltpu.async_copy(
                bufs[b],
                out_hbm.at[pl.ds(base + c * _CHUNK_ROWS, _CHUNK_ROWS)],
                store_sems[b],
            )

        loads = [None] * nbuf
        stores = [None] * nbuf
        store_waited = [True] * nbuf
        for c in range(min(nbuf, nchunk)):
            loads[c] = start_load(c, c)
        for c in range(nchunk):
            b = c % nbuf
            loads[b].wait()
            stores[b] = start_store(c, b)
            store_waited[b] = False
            pc = c - 2
            if pc >= 0 and pc + nbuf < nchunk:
                rb = pc % nbuf
                stores[rb].wait()
                store_waited[rb] = True
                loads[rb] = start_load(pc + nbuf, rb)
        for b in range(nbuf):
            if stores[b] is not None and not store_waited[b]:
                stores[b].wait()

    out = sc_copy(table)
    return out[None]


# speedup vs baseline: 1.0053x; 1.0053x over previous
import functools

import jax
import jax.numpy as jnp
from jax import lax
from jax.experimental import pallas as pl
from jax.experimental.pallas import tpu as pltpu
from jax.experimental.pallas import tpu_sc as plsc

_CHUNK_ROWS = 64


def kernel(seq_len, table):
    del seq_len
    rows, dim = table.shape
    info = plsc.get_sparse_core_info()
    num_workers = info.num_cores * info.num_subcores
    rows_per_worker = rows // num_workers
    nchunk = rows_per_worker // _CHUNK_ROWS

    mesh = plsc.VectorSubcoreMesh(core_axis_name="c", subcore_axis_name="s")

    @functools.partial(
        pl.kernel,
        mesh=mesh,
        out_type=jax.ShapeDtypeStruct((rows, dim), table.dtype),
        scratch_types=[
            pltpu.VMEM((_CHUNK_ROWS, dim), jnp.float32),
            pltpu.VMEM((_CHUNK_ROWS, dim), jnp.float32),
            pltpu.SemaphoreType.DMA,
            pltpu.SemaphoreType.DMA,
            pltpu.SemaphoreType.DMA,
            pltpu.SemaphoreType.DMA,
        ],
    )
    def sc_copy(table_hbm, out_hbm, buf0, buf1, li0, li1, so0, so1):
        wid = lax.axis_index("c") * info.num_subcores + lax.axis_index("s")
        base = wid * rows_per_worker
        bufs = (buf0, buf1)
        load_sems = (li0, li1)
        store_sems = (so0, so1)

        def start_load(c, b):
            return pltpu.async_copy(
                table_hbm.at[pl.ds(base + c * _CHUNK_ROWS, _CHUNK_ROWS)],
                bufs[b],
                load_sems[b],
            )

        def start_store(c, b):
            return pltpu.async_copy(
                bufs[b],
                out_hbm.at[pl.ds(base + c * _CHUNK_ROWS, _CHUNK_ROWS)],
                store_sems[b],
            )

        loads = [None, None]
        stores = [None, None]
        loads[0] = start_load(0, 0)
        for c in range(nchunk):
            b = c & 1
            nb = (c + 1) & 1
            if c + 1 < nchunk:
                if stores[nb] is not None:
                    stores[nb].wait()
                loads[nb] = start_load(c + 1, nb)
            loads[b].wait()
            stores[b] = start_store(c, b)
        for b in range(2):
            if stores[b] is not None:
                stores[b].wait()

    out = sc_copy(table)
    return out[None]


# speedup vs baseline: 1.0205x; 1.0151x over previous
import functools

import jax
import jax.numpy as jnp
from jax import lax
from jax.experimental import pallas as pl
from jax.experimental.pallas import tpu as pltpu
from jax.experimental.pallas import tpu_sc as plsc

_CHUNK_ROWS = 64


def kernel(seq_len, table):
    del seq_len
    rows, dim = table.shape
    info = plsc.get_sparse_core_info()
    num_workers = info.num_cores * info.num_subcores
    rows_per_worker = rows // num_workers
    nchunk = rows_per_worker // _CHUNK_ROWS

    mesh = plsc.VectorSubcoreMesh(core_axis_name="c", subcore_axis_name="s")

    @functools.partial(
        pl.kernel,
        mesh=mesh,
        out_type=jax.ShapeDtypeStruct((rows, dim), table.dtype),
        scratch_types=[
            pltpu.VMEM((_CHUNK_ROWS, dim), jnp.float32),
            pltpu.VMEM((_CHUNK_ROWS, dim), jnp.float32),
            pltpu.SemaphoreType.DMA,
            pltpu.SemaphoreType.DMA,
            pltpu.SemaphoreType.DMA,
            pltpu.SemaphoreType.DMA,
        ],
    )
    def sc_copy(table_hbm, out_hbm, buf0, buf1, li0, li1, so0, so1):
        wid = lax.axis_index("s") * info.num_cores + lax.axis_index("c")
        base = wid * rows_per_worker
        bufs = (buf0, buf1)
        load_sems = (li0, li1)
        store_sems = (so0, so1)

        def start_load(c, b):
            return pltpu.async_copy(
                table_hbm.at[pl.ds(base + c * _CHUNK_ROWS, _CHUNK_ROWS)],
                bufs[b],
                load_sems[b],
            )

        def start_store(c, b):
            return pltpu.async_copy(
                bufs[b],
                out_hbm.at[pl.ds(base + c * _CHUNK_ROWS, _CHUNK_ROWS)],
                store_sems[b],
            )

        loads = [None, None]
        stores = [None, None]
        loads[0] = start_load(0, 0)
        for c in range(nchunk):
            b = c & 1
            nb = (c + 1) & 1
            if c + 1 < nchunk:
                if stores[nb] is not None:
                    stores[nb].wait()
                loads[nb] = start_load(c + 1, nb)
            loads[b].wait()
            stores[b] = start_store(c, b)
        for b in range(2):
            if stores[b] is not None:
                stores[b].wait()

    out = sc_copy(table)
    return out[None]
